# baseline (speedup 1.0000x reference)
import jax
import jax.numpy as jnp
from jax.experimental import pallas as pl
from jax.experimental.pallas import tpu as pltpu


def _reparam_t_kernel(mu_ref, ls_ref, eps_ref, wt_ref):
    w = mu_ref[...] + jnp.exp(ls_ref[...]) * eps_ref[...]
    wt_ref[...] = w.astype(jnp.bfloat16).T


def _matmul_bias_kernel(x_ref, wt_ref, mub_ref, lsb_ref, epsb_ref, o_ref):
    xv = x_ref[...].astype(jnp.bfloat16)
    acc = jnp.dot(xv, wt_ref[...], preferred_element_type=jnp.float32)
    bias = mub_ref[...] + jnp.exp(lsb_ref[...]) * epsb_ref[...]
    o_ref[...] = acc + bias


def kernel(x, mu_w, log_sigma_w, eps_w, mu_b, log_sigma_b, eps_b):
    OUT, IN = mu_w.shape
    orig_shape = x.shape
    x2 = x.reshape(-1, IN)
    B = x2.shape[0]

    tn = min(512, OUT)
    tk = min(2048, IN)
    tb = min(1024, B)

    w_t = pl.pallas_call(
        _reparam_t_kernel,
        out_shape=jax.ShapeDtypeStruct((IN, OUT), jnp.bfloat16),
        grid=(OUT // tn, IN // tk),
        in_specs=[
            pl.BlockSpec((tn, tk), lambda n, k: (n, k)),
            pl.BlockSpec((tn, tk), lambda n, k: (n, k)),
            pl.BlockSpec((tn, tk), lambda n, k: (n, k)),
        ],
        out_specs=pl.BlockSpec((tk, tn), lambda n, k: (k, n)),
        compiler_params=pltpu.CompilerParams(
            dimension_semantics=("parallel", "arbitrary")),
    )(mu_w, log_sigma_w, eps_w)

    y = pl.pallas_call(
        _matmul_bias_kernel,
        out_shape=jax.ShapeDtypeStruct((B, OUT), x.dtype),
        grid=(B // tb,),
        in_specs=[
            pl.BlockSpec((tb, IN), lambda i: (i, 0)),
            pl.BlockSpec((IN, OUT), lambda i: (0, 0)),
            pl.BlockSpec((1, OUT), lambda i: (0, 0)),
            pl.BlockSpec((1, OUT), lambda i: (0, 0)),
            pl.BlockSpec((1, OUT), lambda i: (0, 0)),
        ],
        out_specs=pl.BlockSpec((tb, OUT), lambda i: (i, 0)),
        compiler_params=pltpu.CompilerParams(
            dimension_semantics=("parallel",),
            vmem_limit_bytes=60 * 1024 * 1024),
    )(x2, w_t, mu_b.reshape(1, OUT), log_sigma_b.reshape(1, OUT),
      eps_b.reshape(1, OUT))

    return y.reshape(*orig_shape[:-1], OUT)
